# baseline (speedup 1.0000x reference)
import functools

import jax
import jax.numpy as jnp
from jax import lax
from jax.experimental import pallas as pl
from jax.experimental.pallas import tpu as pltpu
from jax.experimental.pallas import tpu_sc as plsc

N = 10000
E = 320000
D = 128
HD = D // 2
NC = 2
NS = 16
NW = NC * NS
EPT = E // NW
CH = 128
NCHA = 160
EPTA = NCHA * CH
EPAD = NS * EPTA - E
NBUF = 5
RPT = 624
TAIL = N - NS * RPT


def _sc_vec_segsum(x0, x1, src3, dst3, ew3, zeros_rows):
    mesh = plsc.VectorSubcoreMesh(core_axis_name="c", subcore_axis_name="s")

    @functools.partial(
        pl.kernel,
        mesh=mesh,
        out_type=jax.ShapeDtypeStruct((NC, N, HD), jnp.float32),
        scratch_types=[
            pltpu.VMEM((NCHA, CH), jnp.int16),
            pltpu.VMEM((NCHA, CH), jnp.int16),
            pltpu.VMEM((EPTA,), jnp.float32),
            [pltpu.VMEM((CH,), jnp.int32) for _ in range(NBUF)],
            [pltpu.VMEM((CH,), jnp.int32) for _ in range(NBUF)],
            [pltpu.VMEM((CH, HD), jnp.float32) for _ in range(NBUF)],
            pltpu.VMEM_SHARED((N, HD), jnp.float32),
            [pltpu.SemaphoreType.DMA for _ in range(NBUF)],
            [pltpu.SemaphoreType.DMA for _ in range(NBUF)],
        ],
        compiler_params=pltpu.CompilerParams(use_tc_tiling_on_sc=False,
                                             needs_layout_passes=False),
    )
    def k(x0_hbm, x1_hbm, src_hbm, dst_hbm, ewf_hbm, zeros_hbm, out_hbm,
          src_v, dst_v, ew_v, swid, dwid, rows_bufs, acc, sems, ssems):
        cid = lax.axis_index("c")
        sid = lax.axis_index("s")

        pltpu.sync_copy(zeros_hbm.at[pl.ds(0, RPT)],
                        acc.at[pl.ds(sid * RPT, RPT)])

        @pl.when(sid == NS - 1)
        def _():
            pltpu.sync_copy(zeros_hbm.at[pl.ds(0, TAIL)],
                            acc.at[pl.ds(NS * RPT, TAIL)])
        pltpu.sync_copy(src_hbm.at[sid], src_v)
        pltpu.sync_copy(dst_hbm.at[sid], dst_v)
        pltpu.sync_copy(ewf_hbm.at[sid], ew_v)
        plsc.subcore_barrier()

        def widen(packed_ref, kk, wbuf):
            for j in range(CH // 32):
                packed = packed_ref[kk, pl.ds(j * 32, 32)]
                v32 = plsc.bitcast(packed, jnp.int32)
                wbuf[pl.ds(j * 32, 16)] = v32 & 0xFFFF
                wbuf[pl.ds(j * 32 + 16, 16)] = lax.shift_right_logical(
                    v32, 16)

        def start_gather(kk, b):
            widen(src_v, kk, swid[b])

            @pl.when(cid == 0)
            def _():
                pltpu.async_copy(x0_hbm.at[swid[b]], rows_bufs[b], sems[b])

            @pl.when(cid == 1)
            def _():
                pltpu.async_copy(x1_hbm.at[swid[b]], rows_bufs[b], sems[b])

        for b in range(NBUF - 2):
            start_gather(b, b)

        def group_body(g, carry):
            for b in range(NBUF):
                kk = g * NBUF + b
                buf = rows_bufs[b]
                pltpu.make_async_copy(x0_hbm.at[swid[b]], buf,
                                      sems[b]).wait()

                nb = (b + NBUF - 2) % NBUF

                @pl.when(kk >= 2)
                def _():
                    pltpu.make_async_copy(
                        rows_bufs[nb], acc.at[dwid[nb]], ssems[nb]).wait()

                @pl.when(kk + NBUF - 2 < NCHA)
                def _():
                    start_gather(kk + NBUF - 2, nb)

                @plsc.parallel_loop(0, CH // 32, unroll=1)
                def _(j):
                    base = kk * CH + j * 32
                    for half in range(2):
                        for l in range(16):
                            edge = base + 2 * l + half
                            wv = plsc.load_gather(
                                ew_v, [jnp.full((16,), edge, jnp.int32)])
                            row = j * 32 + half * 16 + l
                            for f in range(HD // 16):
                                seg = buf[row, pl.ds(f * 16, 16)]
                                buf[row, pl.ds(f * 16, 16)] = seg * wv

                widen(dst_v, kk, dwid[b])
                pltpu.async_copy(buf, acc.at[dwid[b]], ssems[b], add=True)
            return carry

        lax.fori_loop(0, NCHA // NBUF, group_body, 0, unroll=False)
        for kk in (NCHA - 2, NCHA - 1):
            b = kk % NBUF
            pltpu.make_async_copy(rows_bufs[b], acc.at[dwid[b]],
                                  ssems[b]).wait()
        plsc.subcore_barrier()
        pltpu.sync_copy(acc.at[pl.ds(sid * RPT, RPT)],
                        out_hbm.at[cid, pl.ds(sid * RPT, RPT)])

        @pl.when(sid == NS - 1)
        def _():
            pltpu.sync_copy(acc.at[pl.ds(NS * RPT, TAIL)],
                            out_hbm.at[cid, pl.ds(NS * RPT, TAIL)])

    return k(x0, x1, src3, dst3, ew3, zeros_rows)


def _sc_scalar_segsum(s, src2, dst2, ew2):
    mesh = plsc.VectorSubcoreMesh(core_axis_name="c", subcore_axis_name="s")

    @functools.partial(
        pl.kernel,
        mesh=mesh,
        out_type=jax.ShapeDtypeStruct((NW, N), jnp.float32),
        scratch_types=[
            pltpu.VMEM((N,), jnp.float32),
            pltpu.VMEM((N,), jnp.float32),
            pltpu.VMEM((EPT,), jnp.int32),
            pltpu.VMEM((EPT,), jnp.int32),
            pltpu.VMEM((EPT,), jnp.float32),
        ],
        compiler_params=pltpu.CompilerParams(needs_layout_passes=False),
    )
    def k(s_hbm, src_hbm, dst_hbm, ew_hbm, out_hbm,
          s_v, m_v, src_v, dst_v, ew_v):
        cid = lax.axis_index("c")
        sid = lax.axis_index("s")
        wid = sid * NC + cid

        pltpu.sync_copy(s_hbm, s_v)
        pltpu.sync_copy(src_hbm.at[wid], src_v)
        pltpu.sync_copy(dst_hbm.at[wid], dst_v)
        pltpu.sync_copy(ew_hbm.at[wid], ew_v)

        def zero_body(i, c):
            m_v[pl.ds(i * 16, 16)] = jnp.zeros((16,), jnp.float32)
            return c

        lax.fori_loop(0, N // 16, zero_body, 0, unroll=False)

        def edge_body(i, c):
            sidx = src_v[pl.ds(i * 16, 16)]
            didx = dst_v[pl.ds(i * 16, 16)]
            ww = ew_v[pl.ds(i * 16, 16)]
            sv = plsc.load_gather(s_v, [sidx])
            plsc.addupdate_scatter(m_v, [didx], sv * ww)
            return c

        lax.fori_loop(0, EPT // 16, edge_body, 0, unroll=False)
        pltpu.sync_copy(m_v, out_hbm.at[wid])

    return k(s, src2, dst2, ew2)


def _tc_layer1(agg_halves, x, W1_relT, W1_rootT, b1, W2_rel, W2_root):
    R = 400

    def body(p_r, x_r, wrelT_r, wrootT_r, b1_r, w2rel_r, w2root_r, sr_r):
        p0 = p_r[0]
        p1 = p_r[1]
        w = wrelT_r[...]
        h = jnp.dot(p0, w[:HD, :], preferred_element_type=jnp.float32)
        h = h + jnp.dot(p1, w[HD:, :], preferred_element_type=jnp.float32)
        h = h + jnp.dot(x_r[...], wrootT_r[...],
                        preferred_element_type=jnp.float32)
        h = h + b1_r[...]
        h = jnp.maximum(h, 0.0)
        vrel = jnp.mean(w2rel_r[...], axis=0)
        vroot = jnp.mean(w2root_r[...], axis=0)
        s_col = jnp.sum(h * vrel[None, :], axis=1, keepdims=True)
        r_col = jnp.sum(h * vroot[None, :], axis=1, keepdims=True)
        sr_r[...] = jnp.concatenate([s_col, r_col], axis=1)

    full_spec = pl.BlockSpec((D, D), lambda i: (0, 0))
    return pl.pallas_call(
        body,
        grid=(N // R,),
        in_specs=[pl.BlockSpec((NC, R, HD), lambda i: (0, i, 0)),
                  pl.BlockSpec((R, D), lambda i: (i, 0)),
                  full_spec, full_spec,
                  pl.BlockSpec((1, D), lambda i: (0, 0)),
                  full_spec, full_spec],
        out_specs=pl.BlockSpec((R, 2), lambda i: (i, 0)),
        out_shape=jax.ShapeDtypeStruct((N, 2), jnp.float32),
    )(agg_halves, x, W1_relT, W1_rootT, b1, W2_rel, W2_root)


def _tc_head(m_parts3, r8, b2_rel2, l1T, l1b, l2T, l2b, l3T, l3b, l4T, l4b,
             l5T, l5b):
    def body(m_r, r_r, b2_r, w1, bb1, w2, bb2, w3, bb3, w4, bb4, w5, bb5,
             out_r):
        m = jnp.sum(m_r[...], axis=0)
        g = m + r_r[...] + jnp.mean(b2_r[...])
        h = jnp.maximum(jnp.dot(g, w1[...],
                                preferred_element_type=jnp.float32) + bb1[...], 0.0)
        h = jnp.maximum(jnp.dot(h, w2[...],
                                preferred_element_type=jnp.float32) + bb2[...], 0.0)
        h = jnp.maximum(jnp.dot(h, w3[...],
                                preferred_element_type=jnp.float32) + bb3[...], 0.0)
        h = jnp.maximum(jnp.dot(h, w4[...],
                                preferred_element_type=jnp.float32) + bb4[...], 0.0)
        h = jnp.dot(h, w5[...], preferred_element_type=jnp.float32) + bb5[...]
        mx = jnp.max(h, axis=1, keepdims=True)
        ex = jnp.exp(h - mx)
        lse = mx + jnp.log(jnp.sum(ex, axis=1, keepdims=True))
        out_r[...] = h - lse

    args = (m_parts3, r8, b2_rel2, l1T, l1b, l2T, l2b, l3T, l3b, l4T, l4b,
            l5T, l5b)
    return pl.pallas_call(
        body,
        out_shape=jax.ShapeDtypeStruct((8, 10), jnp.float32),
    )(*args)


def kernel(x, edge_index, batch, edge_weight, W1_rel, b1_rel, W1_root,
           W2_rel, b2_rel, W2_root, lin1_W, lin1_b, lin2_W, lin2_b,
           lin3_W, lin3_b, lin4_W, lin4_b, lin5_W, lin5_b):
    src = edge_index[0]
    dst = edge_index[1]
    ipad = jnp.zeros((EPAD,), jnp.int32)
    src3 = jnp.concatenate([src, ipad]).astype(jnp.int16).reshape(
        NS, NCHA, CH)
    dst3 = jnp.concatenate([dst, ipad]).astype(jnp.int16).reshape(
        NS, NCHA, CH)
    ew3 = jnp.concatenate([edge_weight, jnp.zeros((EPAD,), jnp.float32)]
                          ).reshape(NS, EPTA)
    src2 = src.reshape(NW, EPT)
    dst2 = dst.reshape(NW, EPT)
    ew2 = edge_weight.reshape(NW, EPT)
    zeros_rows = jnp.zeros((RPT, HD), jnp.float32)
    x0 = x[:, :HD]
    x1 = x[:, HD:]

    agg_halves = _sc_vec_segsum(x0, x1, src3, dst3, ew3, zeros_rows)

    sr = _tc_layer1(agg_halves, x,
                    W1_rel.T, W1_root.T, b1_rel.reshape(1, D),
                    W2_rel, W2_root)
    s = sr[:, 0]
    r8 = sr[:, 1].reshape(8, 1250)

    m_parts = _sc_scalar_segsum(s, src2, dst2, ew2)

    out = _tc_head(m_parts.reshape(NW, 8, 1250), r8, b2_rel.reshape(1, D),
                   lin1_W.T, lin1_b.reshape(1, -1),
                   lin2_W.T, lin2_b.reshape(1, -1),
                   lin3_W.T, lin3_b.reshape(1, -1),
                   lin4_W.T, lin4_b.reshape(1, -1),
                   lin5_W.T, lin5_b.reshape(1, -1))
    return out


# speedup vs baseline: 1.7585x; 1.7585x over previous
import functools

import jax
import jax.numpy as jnp
from jax import lax
from jax.experimental import pallas as pl
from jax.experimental.pallas import tpu as pltpu
from jax.experimental.pallas import tpu_sc as plsc

N = 10000
E = 320000
D = 128
HD = D // 2
NC = 2
NS = 16
NW = NC * NS
EPT = E // NW
CH = 80
NCHA = 250
EPTA = NCHA * CH
EPAD = NS * EPTA - E
NBUF = 5
RPT = 624
TAIL = N - NS * RPT


def _sc_vec_segsum(x0, x1, src3, dst3, ew3, zeros_rows):
    mesh = plsc.VectorSubcoreMesh(core_axis_name="c", subcore_axis_name="s")

    @functools.partial(
        pl.kernel,
        mesh=mesh,
        out_type=jax.ShapeDtypeStruct((NC, N, HD), jnp.float32),
        scratch_types=[
            pltpu.VMEM((NCHA, CH), jnp.int32),
            pltpu.VMEM((NCHA, CH), jnp.int32),
            pltpu.VMEM((EPTA,), jnp.float32),
            [pltpu.VMEM((CH, HD), jnp.float32) for _ in range(NBUF)],
            pltpu.VMEM_SHARED((N, HD), jnp.float32),
            [pltpu.SemaphoreType.DMA for _ in range(NBUF)],
            [pltpu.SemaphoreType.DMA for _ in range(NBUF)],
        ],
        compiler_params=pltpu.CompilerParams(use_tc_tiling_on_sc=False,
                                             needs_layout_passes=False),
    )
    def k(x0_hbm, x1_hbm, src_hbm, dst_hbm, ewf_hbm, zeros_hbm, out_hbm,
          src_v, dst_v, ew_v, rows_bufs, acc, sems, ssems):
        cid = lax.axis_index("c")
        sid = lax.axis_index("s")

        pltpu.sync_copy(zeros_hbm.at[pl.ds(0, RPT)],
                        acc.at[pl.ds(sid * RPT, RPT)])

        @pl.when(sid == NS - 1)
        def _():
            pltpu.sync_copy(zeros_hbm.at[pl.ds(0, TAIL)],
                            acc.at[pl.ds(NS * RPT, TAIL)])
        pltpu.sync_copy(src_hbm.at[sid], src_v)
        pltpu.sync_copy(dst_hbm.at[sid], dst_v)
        pltpu.sync_copy(ewf_hbm.at[sid], ew_v)
        plsc.subcore_barrier()

        def start_gather(kk, b):
            @pl.when(cid == 0)
            def _():
                pltpu.async_copy(x0_hbm.at[src_v.at[kk]], rows_bufs[b],
                                 sems[b])

            @pl.when(cid == 1)
            def _():
                pltpu.async_copy(x1_hbm.at[src_v.at[kk]], rows_bufs[b],
                                 sems[b])

        for b in range(NBUF - 2):
            start_gather(b, b)

        def group_body(g, carry):
            for b in range(NBUF):
                kk = g * NBUF + b
                buf = rows_bufs[b]
                pltpu.make_async_copy(x0_hbm.at[src_v.at[kk]], buf,
                                      sems[b]).wait()

                nb = (b + NBUF - 2) % NBUF

                @pl.when(kk >= 2)
                def _():
                    pltpu.make_async_copy(
                        rows_bufs[nb], acc.at[dst_v.at[kk]], ssems[nb]).wait()

                @pl.when(kk + NBUF - 2 < NCHA)
                def _():
                    start_gather(kk + NBUF - 2, nb)

                @plsc.parallel_loop(0, CH // 16, unroll=1)
                def _(j):
                    base = kk * CH + j * 16
                    for l in range(16):
                        wv = plsc.load_gather(
                            ew_v, [jnp.full((16,), base + l, jnp.int32)])
                        e = j * 16 + l
                        for f in range(HD // 16):
                            seg = buf[e, pl.ds(f * 16, 16)]
                            buf[e, pl.ds(f * 16, 16)] = seg * wv

                pltpu.async_copy(buf, acc.at[dst_v.at[kk]], ssems[b],
                                 add=True)
            return carry

        lax.fori_loop(0, NCHA // NBUF, group_body, 0, unroll=False)
        for kk in (NCHA - 2, NCHA - 1):
            b = kk % NBUF
            pltpu.make_async_copy(rows_bufs[b], acc.at[dst_v.at[kk]],
                                  ssems[b]).wait()
        plsc.subcore_barrier()
        pltpu.sync_copy(acc.at[pl.ds(sid * RPT, RPT)],
                        out_hbm.at[cid, pl.ds(sid * RPT, RPT)])

        @pl.when(sid == NS - 1)
        def _():
            pltpu.sync_copy(acc.at[pl.ds(NS * RPT, TAIL)],
                            out_hbm.at[cid, pl.ds(NS * RPT, TAIL)])

    return k(x0, x1, src3, dst3, ew3, zeros_rows)


def _sc_scalar_segsum(s, src2, dst2, ew2):
    mesh = plsc.VectorSubcoreMesh(core_axis_name="c", subcore_axis_name="s")

    @functools.partial(
        pl.kernel,
        mesh=mesh,
        out_type=jax.ShapeDtypeStruct((NW, N), jnp.float32),
        scratch_types=[
            pltpu.VMEM((N,), jnp.float32),
            pltpu.VMEM((N,), jnp.float32),
            pltpu.VMEM((EPT,), jnp.int32),
            pltpu.VMEM((EPT,), jnp.int32),
            pltpu.VMEM((EPT,), jnp.float32),
        ],
        compiler_params=pltpu.CompilerParams(needs_layout_passes=False),
    )
    def k(s_hbm, src_hbm, dst_hbm, ew_hbm, out_hbm,
          s_v, m_v, src_v, dst_v, ew_v):
        cid = lax.axis_index("c")
        sid = lax.axis_index("s")
        wid = sid * NC + cid

        pltpu.sync_copy(s_hbm, s_v)
        pltpu.sync_copy(src_hbm.at[wid], src_v)
        pltpu.sync_copy(dst_hbm.at[wid], dst_v)
        pltpu.sync_copy(ew_hbm.at[wid], ew_v)

        @plsc.parallel_loop(0, N // 16, unroll=1)
        def _(i):
            m_v[pl.ds(i * 16, 16)] = jnp.zeros((16,), jnp.float32)

        @plsc.parallel_loop(0, EPT // 16, unroll=1)
        def _(i):
            sidx = src_v[pl.ds(i * 16, 16)]
            didx = dst_v[pl.ds(i * 16, 16)]
            ww = ew_v[pl.ds(i * 16, 16)]
            sv = plsc.load_gather(s_v, [sidx])
            plsc.addupdate_scatter(m_v, [didx], sv * ww)
        pltpu.sync_copy(m_v, out_hbm.at[wid])

    return k(s, src2, dst2, ew2)


def _tc_layer1(agg_halves, x, W1_relT, W1_rootT, b1, W2_rel, W2_root):
    R = 400

    def body(p_r, x_r, wrelT_r, wrootT_r, b1_r, w2rel_r, w2root_r, sr_r):
        p0 = p_r[0]
        p1 = p_r[1]
        w = wrelT_r[...]
        h = jnp.dot(p0, w[:HD, :], preferred_element_type=jnp.float32)
        h = h + jnp.dot(p1, w[HD:, :], preferred_element_type=jnp.float32)
        h = h + jnp.dot(x_r[...], wrootT_r[...],
                        preferred_element_type=jnp.float32)
        h = h + b1_r[...]
        h = jnp.maximum(h, 0.0)
        vrel = jnp.mean(w2rel_r[...], axis=0)
        vroot = jnp.mean(w2root_r[...], axis=0)
        s_col = jnp.sum(h * vrel[None, :], axis=1, keepdims=True)
        r_col = jnp.sum(h * vroot[None, :], axis=1, keepdims=True)
        sr_r[...] = jnp.concatenate([s_col, r_col], axis=1)

    full_spec = pl.BlockSpec((D, D), lambda i: (0, 0))
    return pl.pallas_call(
        body,
        grid=(N // R,),
        in_specs=[pl.BlockSpec((NC, R, HD), lambda i: (0, i, 0)),
                  pl.BlockSpec((R, D), lambda i: (i, 0)),
                  full_spec, full_spec,
                  pl.BlockSpec((1, D), lambda i: (0, 0)),
                  full_spec, full_spec],
        out_specs=pl.BlockSpec((R, 2), lambda i: (i, 0)),
        out_shape=jax.ShapeDtypeStruct((N, 2), jnp.float32),
    )(agg_halves, x, W1_relT, W1_rootT, b1, W2_rel, W2_root)


def _tc_head(m_parts3, r8, b2_rel2, l1T, l1b, l2T, l2b, l3T, l3b, l4T, l4b,
             l5T, l5b):
    def body(m_r, r_r, b2_r, w1, bb1, w2, bb2, w3, bb3, w4, bb4, w5, bb5,
             out_r):
        m = jnp.sum(m_r[...], axis=0)
        g = m + r_r[...] + jnp.mean(b2_r[...])
        h = jnp.maximum(jnp.dot(g, w1[...],
                                preferred_element_type=jnp.float32) + bb1[...], 0.0)
        h = jnp.maximum(jnp.dot(h, w2[...],
                                preferred_element_type=jnp.float32) + bb2[...], 0.0)
        h = jnp.maximum(jnp.dot(h, w3[...],
                                preferred_element_type=jnp.float32) + bb3[...], 0.0)
        h = jnp.maximum(jnp.dot(h, w4[...],
                                preferred_element_type=jnp.float32) + bb4[...], 0.0)
        h = jnp.dot(h, w5[...], preferred_element_type=jnp.float32) + bb5[...]
        mx = jnp.max(h, axis=1, keepdims=True)
        ex = jnp.exp(h - mx)
        lse = mx + jnp.log(jnp.sum(ex, axis=1, keepdims=True))
        out_r[...] = h - lse

    args = (m_parts3, r8, b2_rel2, l1T, l1b, l2T, l2b, l3T, l3b, l4T, l4b,
            l5T, l5b)
    return pl.pallas_call(
        body,
        out_shape=jax.ShapeDtypeStruct((8, 10), jnp.float32),
    )(*args)


def kernel(x, edge_index, batch, edge_weight, W1_rel, b1_rel, W1_root,
           W2_rel, b2_rel, W2_root, lin1_W, lin1_b, lin2_W, lin2_b,
           lin3_W, lin3_b, lin4_W, lin4_b, lin5_W, lin5_b):
    src = edge_index[0]
    dst = edge_index[1]
    ipad = jnp.zeros((EPAD,), jnp.int32)
    src3 = jnp.concatenate([src, ipad]).reshape(NS, NCHA, CH)
    dst3 = jnp.concatenate([dst, ipad]).reshape(NS, NCHA, CH)
    ew3 = jnp.concatenate([edge_weight, jnp.zeros((EPAD,), jnp.float32)]
                          ).reshape(NS, EPTA)
    src2 = src.reshape(NW, EPT)
    dst2 = dst.reshape(NW, EPT)
    ew2 = edge_weight.reshape(NW, EPT)
    zeros_rows = jnp.zeros((RPT, HD), jnp.float32)
    x0 = x[:, :HD]
    x1 = x[:, HD:]

    agg_halves = _sc_vec_segsum(x0, x1, src3, dst3, ew3, zeros_rows)

    sr = _tc_layer1(agg_halves, x,
                    W1_rel.T, W1_root.T, b1_rel.reshape(1, D),
                    W2_rel, W2_root)
    s = sr[:, 0]
    r8 = sr[:, 1].reshape(8, 1250)

    m_parts = _sc_scalar_segsum(s, src2, dst2, ew2)

    out = _tc_head(m_parts.reshape(NW, 8, 1250), r8, b2_rel.reshape(1, D),
                   lin1_W.T, lin1_b.reshape(1, -1),
                   lin2_W.T, lin2_b.reshape(1, -1),
                   lin3_W.T, lin3_b.reshape(1, -1),
                   lin4_W.T, lin4_b.reshape(1, -1),
                   lin5_W.T, lin5_b.reshape(1, -1))
    return out
